# baseline (speedup 1.0000x reference)
import jax
import jax.numpy as jnp
from jax.experimental import pallas as pl
from jax.experimental.pallas import tpu as pltpu

_TI = 128
_D = 128
_HG = 2


def _emd_block(yi_ref, yjr_ref, x_ref, out_ref, bc_ref):
    i0 = pl.program_id(0)
    n_j = yjr_ref.shape[2]

    @pl.when(i0 == 0)
    def _init():
        out_ref[...] = jnp.zeros_like(out_ref)

    yi = yi_ref[...]
    for d in range(_D):
        tile = jnp.broadcast_to(yi[:, d : d + 1], (_TI, 128))
        bc_ref[pl.ds(d * 8, 8), :, :] = tile.reshape(8, 16, 128)

    part = jnp.zeros((16, 128), jnp.float32)
    for g in range(n_j // (128 * _HG)):
        for c in range(_D // 16):
            acc = [jnp.zeros((8, 16, 128), jnp.bfloat16) for _ in range(_HG)]
            for d in range(c * 16, (c + 1) * 16):
                bct = bc_ref[pl.ds(d * 8, 8), :, :]
                for k in range(_HG):
                    h = g * _HG + k
                    row = yjr_ref[d, :, h * 128 : (h + 1) * 128]
                    acc[k] = acc[k] + jnp.abs(bct - row[None])
            for k in range(_HG):
                h = g * _HG + k
                xw = x_ref[:, h * 128 : (h + 1) * 128].reshape(8, 16, 128)
                part = part + (acc[k].astype(jnp.float32) * xw).sum(axis=0)
    out_ref[...] += jnp.sum(part).reshape(1, 1)


@jax.jit
def kernel(y_i, y_j, x_ij):
    n_i, d = y_i.shape
    n_j = y_j.shape[0]
    yjt = y_j.T.astype(jnp.bfloat16)
    yjr = jnp.broadcast_to(yjt[:, None, :], (d, 16, n_j))
    yi16 = y_i.astype(jnp.bfloat16)

    out = pl.pallas_call(
        _emd_block,
        grid=(n_i // _TI,),
        in_specs=[
            pl.BlockSpec((_TI, d), lambda i: (i, 0)),
            pl.BlockSpec((d, 16, n_j), lambda i: (0, 0, 0)),
            pl.BlockSpec((_TI, n_j), lambda i: (i, 0)),
        ],
        out_specs=pl.BlockSpec((1, 1), lambda i: (0, 0)),
        out_shape=jax.ShapeDtypeStruct((1, 1), jnp.float32),
        scratch_shapes=[pltpu.VMEM((_D * 8, 16, 128), jnp.bfloat16)],
    )(yi16, yjr, x_ij)
    return out[0, 0]


# speedup vs baseline: 1.0903x; 1.0903x over previous
import jax
import jax.numpy as jnp
from jax.experimental import pallas as pl
from jax.experimental.pallas import tpu as pltpu

_TI = 128
_D = 128
_CH = 32


def _emd_block(yi_ref, yjr_ref, x_ref, out_ref, bc_ref):
    i0 = pl.program_id(0)
    n_j = yjr_ref.shape[2]

    @pl.when(i0 == 0)
    def _init():
        out_ref[...] = jnp.zeros_like(out_ref)

    yi = yi_ref[...]
    for d in range(_D):
        tile = jnp.broadcast_to(yi[:, d : d + 1], (_TI, 128))
        bc_ref[pl.ds(d * 8, 8), :, :] = tile.reshape(8, 16, 128)

    x = x_ref[...]
    part = jnp.zeros((8, 128), jnp.float32)
    for h in range(n_j // 128):
        accf = jnp.zeros((8, 16, 128), jnp.float32)
        for c in range(_D // _CH):
            acc = jnp.zeros((8, 16, 128), jnp.bfloat16)
            for d in range(c * _CH, (c + 1) * _CH):
                row = yjr_ref[d, :, h * 128 : (h + 1) * 128]
                acc = acc + jnp.abs(bc_ref[pl.ds(d * 8, 8), :, :] - row[None])
            accf = accf + acc.astype(jnp.float32)
        w = accf.reshape(_TI, 128) * x[:, h * 128 : (h + 1) * 128]
        part = part + w.reshape(16, 8, 128).sum(axis=0)
    out_ref[...] += jnp.sum(part).reshape(1, 1)


@jax.jit
def kernel(y_i, y_j, x_ij):
    n_i, d = y_i.shape
    n_j = y_j.shape[0]
    yjt = y_j.T.astype(jnp.bfloat16)
    yjr = jnp.broadcast_to(yjt[:, None, :], (d, 16, n_j))
    yi16 = y_i.astype(jnp.bfloat16)

    out = pl.pallas_call(
        _emd_block,
        grid=(n_i // _TI,),
        in_specs=[
            pl.BlockSpec((_TI, d), lambda i: (i, 0)),
            pl.BlockSpec((d, 16, n_j), lambda i: (0, 0, 0)),
            pl.BlockSpec((_TI, n_j), lambda i: (i, 0)),
        ],
        out_specs=pl.BlockSpec((1, 1), lambda i: (0, 0)),
        out_shape=jax.ShapeDtypeStruct((1, 1), jnp.float32),
        scratch_shapes=[pltpu.VMEM((_D * 8, 16, 128), jnp.bfloat16)],
    )(yi16, yjr, x_ij)
    return out[0, 0]
